# Initial kernel scaffold; baseline (speedup 1.0000x reference)
import jax
import jax.numpy as jnp
from jax.experimental import pallas as pl


def kernel(x, edge_index, edge_attr, u, batch, W1, b1, W2, b2, ln_w, ln_b):
    raise NotImplementedError("write your pallas kernel here")


# speedup vs baseline: 12.1671x; 12.1671x over previous
import functools

import jax
import jax.numpy as jnp
from jax.experimental import pallas as pl

N = 10000
E = 320000
B = 256
ND = 128
ED = 16
GD = 128
HID = 4 * GD

_NBLK = 1000
_EBLK = 4000


def _bounds_body(batch_ref, lo_ref, hi_ref):
    biota = jax.lax.broadcasted_iota(jnp.int32, (B, 1), 0)
    lo = jnp.zeros((B, 1), jnp.float32)
    hi = jnp.zeros((B, 1), jnp.float32)
    nchunk = 8
    c = 1280

    def step(i, carry):
        lo, hi = carry
        seg = batch_ref[0, pl.ds(i * c, c)][None, :]
        lo = lo + jnp.sum((seg < biota).astype(jnp.float32), axis=1, keepdims=True)
        hi = hi + jnp.sum((seg <= biota).astype(jnp.float32), axis=1, keepdims=True)
        return lo, hi

    lo, hi = jax.lax.fori_loop(0, nchunk, step, (lo, hi))
    lo_ref[...] = lo.astype(jnp.int32)
    hi_ref[...] = hi.astype(jnp.int32)


def _xsum_body(x_ref, lo_ref, hi_ref, out_ref):
    k = pl.program_id(0)
    gi = jax.lax.broadcasted_iota(jnp.int32, (1, _NBLK), 1) + k * _NBLK
    oh = ((gi >= lo_ref[...]) & (gi < hi_ref[...])).astype(jnp.float32)
    part = jnp.dot(oh, x_ref[...], preferred_element_type=jnp.float32)

    @pl.when(k == 0)
    def _():
        out_ref[...] = jnp.zeros_like(out_ref)

    out_ref[...] += part


def _esum_body(src_ref, attr_ref, lo_ref, hi_ref, esum_ref, ecnt_ref):
    k = pl.program_id(0)
    src = src_ref[0, 0, :][None, :]
    oh = ((src >= lo_ref[...]) & (src < hi_ref[...])).astype(jnp.float32)
    part = jnp.dot(oh, attr_ref[...], preferred_element_type=jnp.float32)
    cnt = jnp.sum(oh, axis=1, keepdims=True)

    @pl.when(k == 0)
    def _():
        esum_ref[...] = jnp.zeros_like(esum_ref)
        ecnt_ref[...] = jnp.zeros_like(ecnt_ref)

    esum_ref[...] += part
    ecnt_ref[...] += cnt


def _mlp_body(esum_ref, ecnt_ref, xsum_ref, lo_ref, hi_ref, u_ref,
              W1_ref, b1_ref, W2_ref, b2_ref, lnw_ref, lnb_ref, out_ref):
    ecnt = jnp.maximum(ecnt_ref[...], 1.0)
    xcnt = jnp.maximum((hi_ref[...] - lo_ref[...]).astype(jnp.float32), 1.0)
    e_mean = esum_ref[...] / ecnt
    x_mean = xsum_ref[...] / xcnt
    u = u_ref[...]
    h = (jnp.dot(u, W1_ref[0:GD, :], preferred_element_type=jnp.float32)
         + jnp.dot(e_mean, W1_ref[GD:GD + ED, :], preferred_element_type=jnp.float32)
         + jnp.dot(x_mean, W1_ref[GD + ED:GD + ED + ND, :], preferred_element_type=jnp.float32)
         + b1_ref[...])
    h = jnp.maximum(h, 0.0)
    o = jnp.dot(h, W2_ref[...], preferred_element_type=jnp.float32) + b2_ref[...] + u
    mu = jnp.mean(o, axis=-1, keepdims=True)
    d = o - mu
    var = jnp.mean(d * d, axis=-1, keepdims=True)
    out_ref[...] = d * jax.lax.rsqrt(var + 1e-5) * lnw_ref[...] + lnb_ref[...]


def kernel(x, edge_index, edge_attr, u, batch, W1, b1, W2, b2, ln_w, ln_b):
    batch2 = jnp.pad(batch.reshape(1, N), ((0, 0), (0, 240)), constant_values=B)
    lo, hi = pl.pallas_call(
        _bounds_body,
        out_shape=(jax.ShapeDtypeStruct((B, 1), jnp.int32),
                   jax.ShapeDtypeStruct((B, 1), jnp.int32)),
    )(batch2)

    xsum = pl.pallas_call(
        _xsum_body,
        grid=(N // _NBLK,),
        in_specs=[
            pl.BlockSpec((_NBLK, ND), lambda k: (k, 0)),
            pl.BlockSpec((B, 1), lambda k: (0, 0)),
            pl.BlockSpec((B, 1), lambda k: (0, 0)),
        ],
        out_specs=pl.BlockSpec((B, ND), lambda k: (0, 0)),
        out_shape=jax.ShapeDtypeStruct((B, ND), jnp.float32),
    )(x, lo, hi)

    src3 = edge_index[0].reshape(E // _EBLK, 1, _EBLK)
    esum, ecnt = pl.pallas_call(
        _esum_body,
        grid=(E // _EBLK,),
        in_specs=[
            pl.BlockSpec((1, 1, _EBLK), lambda k: (k, 0, 0)),
            pl.BlockSpec((_EBLK, ED), lambda k: (k, 0)),
            pl.BlockSpec((B, 1), lambda k: (0, 0)),
            pl.BlockSpec((B, 1), lambda k: (0, 0)),
        ],
        out_specs=(pl.BlockSpec((B, ED), lambda k: (0, 0)),
                   pl.BlockSpec((B, 1), lambda k: (0, 0))),
        out_shape=(jax.ShapeDtypeStruct((B, ED), jnp.float32),
                   jax.ShapeDtypeStruct((B, 1), jnp.float32)),
    )(src3, edge_attr, lo, hi)

    out = pl.pallas_call(
        _mlp_body,
        out_shape=jax.ShapeDtypeStruct((B, GD), jnp.float32),
    )(esum, ecnt, xsum, lo, hi, u,
      W1, b1.reshape(1, HID), W2, b2.reshape(1, GD),
      ln_w.reshape(1, GD), ln_b.reshape(1, GD))
    return out
